# baseline (speedup 1.0000x reference)
import jax
import jax.numpy as jnp
from jax.experimental import pallas as pl
from jax.experimental.pallas import tpu as pltpu


def kernel(x, embed_table):
    b, _, h, w = x.shape
    hw = h * w
    d = embed_table.shape[1]
    rep = 4

    def body(e_ref, o_ref, stage, sem):
        for r in range(rep):
            stage[r] = e_ref[...]
        for i in range(b // rep):
            pltpu.make_async_copy(stage, o_ref.at[pl.ds(rep * i, rep)],
                                  sem).start()
        for i in range(b // rep):
            pltpu.make_async_copy(stage, o_ref.at[pl.ds(rep * i, rep)],
                                  sem).wait()

    out = pl.pallas_call(
        body,
        in_specs=[pl.BlockSpec(memory_space=pltpu.MemorySpace.VMEM)],
        out_specs=pl.BlockSpec(memory_space=pltpu.MemorySpace.HBM),
        out_shape=jax.ShapeDtypeStruct((b, hw, d), embed_table.dtype),
        scratch_shapes=[pltpu.VMEM((rep, hw, d), embed_table.dtype),
                        pltpu.SemaphoreType.DMA],
    )(embed_table)
    return out.reshape(b, h, w, d).transpose(0, 3, 1, 2)
